# baseline (speedup 1.0000x reference)
import functools

import jax
import jax.numpy as jnp
from jax import lax
from jax.experimental import pallas as pl
from jax.experimental.pallas import tpu as pltpu

NPOINT = 512
NSAMPLE = 64
EPS = 1e-5
BLK = 4096
BMW = 128


def _bn_eval(x, g, b, m, v):
    return (x - m[None, :, None]) / jnp.sqrt(v[None, :, None] + EPS) \
        * g[None, :, None] + b[None, :, None]


def _scores(xyz, W1, g1, b1, m1, v1, W2, g2, b2, m2, v2, W3, bias3):
    xyz_trans = jnp.transpose(xyz, (0, 2, 1))
    h = jnp.einsum('oi,bin->bon', W1, xyz_trans)
    h = jax.nn.relu(_bn_eval(h, g1, b1, m1, v1))
    h = jnp.einsum('oi,bin->bon', W2, h)
    h = jax.nn.relu(_bn_eval(h, g2, b2, m2, v2))
    logits = jnp.einsum('oi,bin->bon', W3, h) + bias3[None, :, None]
    return xyz_trans, jax.nn.sigmoid(logits)


def _bm_body(p_ref, bm_ref):
    bm_ref[0, 0] = jnp.max(
        p_ref[0].reshape(NPOINT, BLK // BMW, BMW), axis=-1)


def _block_max(p):
    B, P, N = p.shape
    nblk = N // BLK
    bm = pl.pallas_call(
        _bm_body,
        grid=(B, nblk),
        in_specs=[pl.BlockSpec((1, P, BLK), lambda b, j: (b, 0, j))],
        out_specs=pl.BlockSpec((1, 1, P, BLK // BMW), lambda b, j: (b, j, 0, 0)),
        out_shape=jax.ShapeDtypeStruct((B, nblk, P, BLK // BMW), jnp.float32),
        compiler_params=pltpu.CompilerParams(
            dimension_semantics=("parallel", "parallel")),
    )(p)
    return bm.transpose(0, 2, 1, 3).reshape(B, P, N // BMW)


def kernel(xyz, features, W1, g1, b1, m1, v1, W2, g2, b2, m2, v2, W3, bias3):
    B, N, _ = xyz.shape
    xyz_trans, p = _scores(xyz, W1, g1, b1, m1, v1, W2, g2, b2, m2, v2, W3,
                           bias3)

    bm = _block_max(p)
    _, group_indices = lax.top_k(p, NSAMPLE)

    def group(feat, idx):
        return feat[:, idx]

    grouped_xyz = jax.vmap(group)(xyz_trans, group_indices)
    grouped_feat = jax.vmap(group)(features, group_indices)
    out = jnp.concatenate([grouped_xyz, grouped_feat], axis=1)
    return out + 0.0 * bm[0, 0, 0]


# speedup vs baseline: 11.0986x; 11.0986x over previous
import functools

import jax
import jax.numpy as jnp
from jax import lax
from jax.experimental import pallas as pl
from jax.experimental.pallas import tpu as pltpu
from jax.experimental.pallas import tpu_sc as plsc

NPOINT = 512
NSAMPLE = 64
EPS = 1e-5
B = 2
N = 16384
C = 16
NROW = B * NPOINT
BLK = 4096
BMW = 32
NBM = N // BMW
CAP = 256
NW = 32
ROWS_PW = NROW // NW
NCH = 3 + C
NJOB = B * NCH


def _bn_eval(x, g, b, m, v):
    return (x - m[None, :, None]) / jnp.sqrt(v[None, :, None] + EPS) \
        * g[None, :, None] + b[None, :, None]


def _scores(xyz, W1, g1, b1, m1, v1, W2, g2, b2, m2, v2, W3, bias3):
    xyz_trans = jnp.transpose(xyz, (0, 2, 1))
    h = jnp.einsum('oi,bin->bon', W1, xyz_trans)
    h = jax.nn.relu(_bn_eval(h, g1, b1, m1, v1))
    h = jnp.einsum('oi,bin->bon', W2, h)
    h = jax.nn.relu(_bn_eval(h, g2, b2, m2, v2))
    logits = jnp.einsum('oi,bin->bon', W3, h) + bias3[None, :, None]
    return xyz_trans, jax.nn.sigmoid(logits)


def _bm_body(p_ref, bm_ref):
    bm_ref[0, 0] = jnp.max(
        p_ref[0].reshape(NPOINT, BLK // BMW, BMW), axis=-1)


def _block_max(p):
    nblk = N // BLK
    bm = pl.pallas_call(
        _bm_body,
        grid=(B, nblk),
        in_specs=[pl.BlockSpec((1, NPOINT, BLK), lambda b, j: (b, 0, j))],
        out_specs=pl.BlockSpec((1, 1, NPOINT, BLK // BMW),
                               lambda b, j: (b, j, 0, 0)),
        out_shape=jax.ShapeDtypeStruct((B, nblk, NPOINT, BLK // BMW),
                                       jnp.float32),
        compiler_params=pltpu.CompilerParams(
            dimension_semantics=("parallel", "parallel")),
    )(p)
    return bm.transpose(0, 2, 1, 3).reshape(NROW, NBM)


def _make_collect():
    mesh = plsc.VectorSubcoreMesh(core_axis_name="c", subcore_axis_name="s")

    @functools.partial(
        pl.kernel,
        out_type=[
            jax.ShapeDtypeStruct((NROW, CAP), jnp.float32),
            jax.ShapeDtypeStruct((NROW, CAP), jnp.int32),
        ],
        mesh=mesh,
        compiler_params=pltpu.CompilerParams(needs_layout_passes=False),
        scratch_types=[
            pltpu.VMEM((N,), jnp.float32),
            pltpu.VMEM((ROWS_PW * 16,), jnp.float32),
            pltpu.VMEM((CAP,), jnp.float32),
            pltpu.VMEM((CAP,), jnp.int32),
        ],
    )
    def collect(p_hbm, thr_hbm, val_hbm, idx_hbm, rowbuf, thrbuf, cbuf, ibuf):
        wid = lax.axis_index("s") * 2 + lax.axis_index("c")
        base = wid * ROWS_PW
        pltpu.sync_copy(thr_hbm.at[pl.ds(base * 16, ROWS_PW * 16)], thrbuf)
        iot = lax.iota(jnp.int32, 16)
        neg = jnp.full((16,), -1.0, jnp.float32)

        def row_body(j, _):
            r = base + j
            pltpu.sync_copy(p_hbm.at[r], rowbuf)
            tvec = thrbuf[pl.ds(j * 16, 16)]
            for k in range(CAP // 16):
                cbuf[pl.ds(k * 16, 16)] = neg

            def chunk(i, off):
                v = rowbuf[pl.ds(i * 16, 16)]
                mask = v >= tvec
                cnt = plsc.all_reduce_population_count(mask)[0]
                plsc.store_compressed(cbuf.at[pl.ds(off, 16)], v, mask=mask)
                plsc.store_compressed(ibuf.at[pl.ds(off, 16)], iot + i * 16,
                                      mask=mask)
                return jnp.minimum(off + cnt, CAP - 16)

            lax.fori_loop(0, N // 16, chunk, jnp.int32(0), unroll=4)
            pltpu.sync_copy(cbuf, val_hbm.at[r])
            pltpu.sync_copy(ibuf, idx_hbm.at[r])
            return 0

        lax.fori_loop(0, ROWS_PW, row_body, 0)

    return collect


def _make_gather():
    mesh = plsc.VectorSubcoreMesh(core_axis_name="c", subcore_axis_name="s")
    npts = NPOINT * NSAMPLE

    @functools.partial(
        pl.kernel,
        out_type=jax.ShapeDtypeStruct((NJOB, npts), jnp.float32),
        mesh=mesh,
        compiler_params=pltpu.CompilerParams(needs_layout_passes=False),
        scratch_types=[
            pltpu.VMEM((N,), jnp.float32),
            pltpu.VMEM((npts,), jnp.int32),
            pltpu.VMEM((npts,), jnp.float32),
        ],
    )
    def gather(src_hbm, gidx_hbm, out_hbm, srcbuf, idxbuf, obuf):
        wid = lax.axis_index("s") * 2 + lax.axis_index("c")

        def do_job(job):
            b = job // NCH
            pltpu.sync_copy(src_hbm.at[job], srcbuf)
            pltpu.sync_copy(gidx_hbm.at[b], idxbuf)

            def chunk(i, _):
                idxv = idxbuf[pl.ds(i * 16, 16)]
                obuf[pl.ds(i * 16, 16)] = plsc.load_gather(srcbuf, [idxv])
                return 0

            lax.fori_loop(0, npts // 16, chunk, 0, unroll=8)
            pltpu.sync_copy(obuf, out_hbm.at[job])

        do_job(wid)

        @pl.when(wid < NJOB - NW)
        def _():
            do_job(wid + NW)

    return gather


def kernel(xyz, features, W1, g1, b1, m1, v1, W2, g2, b2, m2, v2, W3, bias3):
    xyz_trans, p = _scores(xyz, W1, g1, b1, m1, v1, W2, g2, b2, m2, v2, W3,
                           bias3)

    bm = _block_max(p)
    t = lax.top_k(bm, NSAMPLE)[0][:, NSAMPLE - 1]
    thr = jnp.broadcast_to(t[:, None], (NROW, 16)).reshape(NROW * 16)

    cval, cidx = _make_collect()(p.reshape(NROW, N), thr)

    pos = lax.top_k(cval, NSAMPLE)[1]
    gidx = jnp.take_along_axis(cidx, pos, axis=1)
    gidx2 = gidx.reshape(B, NPOINT * NSAMPLE)

    src = jnp.concatenate([xyz_trans, features], axis=1)
    out = _make_gather()(src.reshape(NJOB, N), gidx2)
    return out.reshape(B, NCH, NPOINT, NSAMPLE)
